# baseline (device time: 152889 ns/iter reference)
import os

import jax
import jax.numpy as jnp
from jax import lax
from jax.experimental import pallas as pl
from jax.experimental.pallas import tpu as pltpu

N_DEV = 8
HP = 8
DH = 128
SQ = 2048
SKV = 2048
DM = 1024
HC = DM // 2
CH = SQ // N_DEV
QT = 512
NQT = SQ // QT
SCALE = 0.08838834764831843
N_HOPS = 2 * (N_DEV - 1)

_INTERPRET = os.environ.get("SCBAND_INTERPRET") == "1"


def kernel(x, Wq, K_ext, V_ext, Wo):
    xb = x.astype(jnp.bfloat16)
    wqb = Wq.astype(jnp.bfloat16)
    wob = Wo.astype(jnp.bfloat16)

    def body(x_ref, wq_ref, k_hbm, v_hbm, wo_ref, out_ref,
             kstage, vstage, q_ref, ctx_ref, acc_bf, rsb_p, rsb_m,
             bias0, biasb, biasg, kv_sems,
             send_p, recv_p, send_m, recv_m):
        my = lax.axis_index("i")

        kv_copies = [
            pltpu.make_async_copy(
                k_hbm.at[0, :, pl.ds(my * HP, HP), :], kstage, kv_sems.at[0]),
            pltpu.make_async_copy(
                v_hbm.at[0, :, pl.ds(my * HP, HP), :], vstage, kv_sems.at[1]),
        ]
        for c in kv_copies:
            c.start()
        p_ = lax.rem(my, 4)
        cz = my // 4
        cy = p_ // 2
        cx = (p_ % 2) ^ cy

        def ring_idx(x, y, z):
            return 4 * z + 2 * y + (x ^ y)

        xp = ring_idx(1 - cx, cy, cz)
        yp = ring_idx(cx, 1 - cy, cz)
        zp = ring_idx(cx, cy, 1 - cz)
        cid = cx + 2 * cy + 4 * cz

        barrier = pltpu.get_barrier_semaphore()
        for nbr in (xp, yp, zp):
            pl.semaphore_signal(barrier, inc=1, device_id=(nbr,),
                                device_id_type=pl.DeviceIdType.MESH)
        pl.semaphore_wait(barrier, 3)

        for rt in range(NQT):
            r0 = rt * QT
            q_ref[r0:r0 + QT, :] = lax.dot_general(
                x_ref[0, r0:r0 + QT, :], wq_ref[...],
                (((1,), (0,)), ((), ())),
                preferred_element_type=jnp.float32).astype(jnp.bfloat16)

        for c in kv_copies:
            c.wait()

        qi = lax.broadcasted_iota(jnp.int32, (QT, SKV), 0)
        ki = lax.broadcasted_iota(jnp.int32, (QT, SKV), 1)
        mask0 = (jnp.abs(qi - ki) <= 128) | (ki < 32) | (qi < 32)
        bias0[...] = jnp.where(mask0, 0.0, -1e9)

        for j in range(HP):
            c0, c1 = j * DH, (j + 1) * DH
            s = lax.dot_general(
                q_ref[0:QT, c0:c1], kstage[:, j, :].astype(jnp.bfloat16),
                (((1,), (1,)), ((), ())),
                preferred_element_type=jnp.float32) * SCALE + bias0[...]
            w = jnp.exp(s)
            r = 1.0 / jnp.sum(w, axis=1, keepdims=True)
            p = (w * r).astype(jnp.bfloat16)
            ctx_ref[0:QT, c0:c1] = lax.dot_general(
                p, vstage[:, j, :].astype(jnp.bfloat16),
                (((1,), (0,)), ((), ())),
                preferred_element_type=jnp.float32).astype(jnp.bfloat16)

        kig = lax.broadcasted_iota(jnp.int32, (QT, 128), 1)
        biasg[...] = jnp.where(kig < 32, 0.0, -1e9)
        WB = QT + 2 * 128
        for t in range(1, NQT):
            w0 = min(t * QT - 128, SKV - WB)
            qib = lax.broadcasted_iota(jnp.int32, (QT, WB), 0) + t * QT
            kib = lax.broadcasted_iota(jnp.int32, (QT, WB), 1) + w0
            biasb[...] = jnp.where(jnp.abs(qib - kib) <= 128, 0.0, -1e9)

            for j in range(HP):
                c0, c1 = j * DH, (j + 1) * DH
                qh = q_ref[t * QT:(t + 1) * QT, c0:c1]
                sb = lax.dot_general(
                    qh, kstage[w0:w0 + WB, j, :].astype(jnp.bfloat16),
                    (((1,), (1,)), ((), ())),
                    preferred_element_type=jnp.float32) * SCALE + biasb[...]
                sg = lax.dot_general(
                    qh, kstage[0:128, j, :].astype(jnp.bfloat16),
                    (((1,), (1,)), ((), ())),
                    preferred_element_type=jnp.float32) * SCALE + biasg[...]
                wb = jnp.exp(sb)
                wg = jnp.exp(sg)
                r = 1.0 / (jnp.sum(wb, axis=1, keepdims=True)
                           + jnp.sum(wg, axis=1, keepdims=True))
                ctx = lax.dot_general(
                    (wb * r).astype(jnp.bfloat16),
                    vstage[w0:w0 + WB, j, :].astype(jnp.bfloat16),
                    (((1,), (0,)), ((), ())),
                    preferred_element_type=jnp.float32)
                ctx = ctx + lax.dot_general(
                    (wg * r).astype(jnp.bfloat16),
                    vstage[0:128, j, :].astype(jnp.bfloat16),
                    (((1,), (0,)), ((), ())),
                    preferred_element_type=jnp.float32)
                ctx_ref[t * QT:(t + 1) * QT, c0:c1] = ctx.astype(jnp.bfloat16)

        for rt in range(NQT):
            r0 = rt * QT
            acc_bf[r0:r0 + QT, :] = lax.dot_general(
                ctx_ref[r0:r0 + QT, :], wo_ref[...],
                (((1,), (0,)), ((), ())),
                preferred_element_type=jnp.float32).astype(jnp.bfloat16)

        L, R = (0, HC), (HC, DM)

        def exchange(partner, sends, col, sems, rsb=None, base=0):
            c0, c1 = col
            rs = []
            for i, (sid, rid) in enumerate(sends):
                slot = base + i
                dst = (rsb.at[slot] if rsb is not None
                       else acc_bf.at[pl.ds(sid * CH, CH), c0:c1])
                r = pltpu.make_async_remote_copy(
                    src_ref=acc_bf.at[pl.ds(sid * CH, CH), c0:c1],
                    dst_ref=dst,
                    send_sem=sems[0].at[slot], recv_sem=sems[1].at[slot],
                    device_id=(partner,),
                    device_id_type=pl.DeviceIdType.MESH)
                r.start()
                rs.append(r)
            return rs

        def accumulate(sends, col, rsb, base):
            c0, c1 = col
            for i, (_, rid) in enumerate(sends):
                off = rid * CH
                acc_bf[pl.ds(off, CH), c0:c1] = (
                    acc_bf[pl.ds(off, CH), c0:c1] + rsb[base + i])

        b2_ = [(0, 0), (1, 0), (0, 1), (1, 1)]
        rs_L = [
            (xp, [((1 - cx) + 2 * b1 + 4 * b2, cx + 2 * b1 + 4 * b2)
                  for b1, b2 in b2_]),
            (yp, [(cx + 2 * (1 - cy) + 4 * b2, cx + 2 * cy + 4 * b2)
                  for b2 in (0, 1)]),
            (zp, [(cx + 2 * cy + 4 * (1 - cz), cid)]),
        ]
        rs_R = [
            (yp, [(bx + 2 * (1 - cy) + 4 * bz, bx + 2 * cy + 4 * bz)
                  for bx, bz in b2_]),
            (zp, [(bx + 2 * cy + 4 * (1 - cz), bx + 2 * cy + 4 * cz)
                  for bx in (0, 1)]),
            (xp, [((1 - cx) + 2 * cy + 4 * cz, cid)]),
        ]
        ag_L = [
            (zp, [(cid, 0)]),
            (yp, [(cx + 2 * cy + 4 * b2, 0) for b2 in (0, 1)]),
            (xp, [(cx + 2 * b1 + 4 * b2, 0) for b1, b2 in b2_]),
        ]
        ag_R = [
            (xp, [(cid, 0)]),
            (zp, [(bx + 2 * cy + 4 * cz, 0) for bx in (0, 1)]),
            (yp, [(bx + 2 * cy + 4 * bz, 0) for bx, bz in b2_]),
        ]
        bases = [0, 4, 6]
        ag_bases = [7, 8, 10]

        for st in range(3):
            pl_, sl = rs_L[st]
            pm_, sm_ = rs_R[st]
            rl = exchange(pl_, sl, L, (send_p, recv_p), rsb_p, bases[st])
            rm = exchange(pm_, sm_, R, (send_m, recv_m), rsb_m, bases[st])
            for r in rl + rm:
                r.wait()
            accumulate(sl, L, rsb_p, bases[st])
            accumulate(sm_, R, rsb_m, bases[st])

        for st in range(3):
            pl_, sl = ag_L[st]
            pm_, sm_ = ag_R[st]
            rl = exchange(pl_, sl, L, (send_p, recv_p), None, ag_bases[st])
            rm = exchange(pm_, sm_, R, (send_m, recv_m), None, ag_bases[st])
            for r in rl + rm:
                r.wait()

        out_ref[0] = acc_bf[...].astype(jnp.float32)

    kwargs = {}
    if _INTERPRET:
        kwargs["interpret"] = pltpu.InterpretParams()

    return pl.pallas_call(
        body,
        out_shape=jax.ShapeDtypeStruct((1, SQ, DM), jnp.float32),
        in_specs=[
            pl.BlockSpec(memory_space=pltpu.MemorySpace.VMEM),
            pl.BlockSpec(memory_space=pltpu.MemorySpace.VMEM),
            pl.BlockSpec(memory_space=pltpu.MemorySpace.HBM),
            pl.BlockSpec(memory_space=pltpu.MemorySpace.HBM),
            pl.BlockSpec(memory_space=pltpu.MemorySpace.VMEM),
        ],
        out_specs=pl.BlockSpec(memory_space=pltpu.MemorySpace.VMEM),
        scratch_shapes=[
            pltpu.VMEM((SKV, HP, DH), jnp.float32),
            pltpu.VMEM((SKV, HP, DH), jnp.float32),
            pltpu.VMEM((SQ, DM), jnp.bfloat16),
            pltpu.VMEM((SQ, DM), jnp.bfloat16),
            pltpu.VMEM((SQ, DM), jnp.bfloat16),
            pltpu.VMEM((N_DEV - 1, CH, HC), jnp.bfloat16),
            pltpu.VMEM((N_DEV - 1, CH, HC), jnp.bfloat16),
            pltpu.VMEM((QT, SKV), jnp.float32),
            pltpu.VMEM((QT, QT + 256), jnp.float32),
            pltpu.VMEM((QT, 128), jnp.float32),
            pltpu.SemaphoreType.DMA((2,)),
            pltpu.SemaphoreType.DMA((N_HOPS,)),
            pltpu.SemaphoreType.DMA((N_HOPS,)),
            pltpu.SemaphoreType.DMA((N_HOPS,)),
            pltpu.SemaphoreType.DMA((N_HOPS,)),
        ],
        compiler_params=pltpu.CompilerParams(
            collective_id=0,
            vmem_limit_bytes=62 * 1024 * 1024,
        ),
        **kwargs,
    )(xb, wqb, K_ext, V_ext, wob)


# device time: 138755 ns/iter; 1.1019x vs baseline; 1.1019x over previous
import os

import jax
import jax.numpy as jnp
from jax import lax
from jax.experimental import pallas as pl
from jax.experimental.pallas import tpu as pltpu

N_DEV = 8
HP = 8
DH = 128
SQ = 2048
SKV = 2048
DM = 1024
HC = DM // 2
CH = SQ // N_DEV
QT = 512
NQT = SQ // QT
SCALE = 0.08838834764831843
N_HOPS = 2 * (N_DEV - 1)

_INTERPRET = os.environ.get("SCBAND_INTERPRET") == "1"


def kernel(x, Wq, K_ext, V_ext, Wo):
    my_out = lax.axis_index("i")
    xb = x.astype(jnp.bfloat16)
    wqb = Wq.astype(jnp.bfloat16)
    wob = Wo.astype(jnp.bfloat16)
    kb = lax.dynamic_slice(
        K_ext[0], (0, my_out * HP, 0), (SKV, HP, DH)
    ).astype(jnp.bfloat16).reshape(SKV, HP * DH)
    vb = lax.dynamic_slice(
        V_ext[0], (0, my_out * HP, 0), (SKV, HP, DH)
    ).astype(jnp.bfloat16).reshape(SKV, HP * DH)

    def body(x_ref, wq_ref, k_ref, v_ref, wo_ref, out_ref,
             q_ref, ctx_ref, acc_bf, rsb_p, rsb_m,
             bias0, biasb, biasg,
             send_p, recv_p, send_m, recv_m):
        my = lax.axis_index("i")
        p_ = lax.rem(my, 4)
        cz = my // 4
        cy = p_ // 2
        cx = (p_ % 2) ^ cy

        def ring_idx(x, y, z):
            return 4 * z + 2 * y + (x ^ y)

        xp = ring_idx(1 - cx, cy, cz)
        yp = ring_idx(cx, 1 - cy, cz)
        zp = ring_idx(cx, cy, 1 - cz)
        cid = cx + 2 * cy + 4 * cz

        barrier = pltpu.get_barrier_semaphore()
        for nbr in (xp, yp, zp):
            pl.semaphore_signal(barrier, inc=1, device_id=(nbr,),
                                device_id_type=pl.DeviceIdType.MESH)
        pl.semaphore_wait(barrier, 3)

        for rt in range(NQT):
            r0 = rt * QT
            q_ref[r0:r0 + QT, :] = lax.dot_general(
                x_ref[0, r0:r0 + QT, :], wq_ref[...],
                (((1,), (0,)), ((), ())),
                preferred_element_type=jnp.float32).astype(jnp.bfloat16)

        qi = lax.broadcasted_iota(jnp.int32, (QT, SKV), 0)
        ki = lax.broadcasted_iota(jnp.int32, (QT, SKV), 1)
        mask0 = (jnp.abs(qi - ki) <= 128) | (ki < 32) | (qi < 32)
        bias0[...] = jnp.where(mask0, 0.0, -1e9)

        for j in range(HP):
            c0, c1 = j * DH, (j + 1) * DH
            s = lax.dot_general(
                q_ref[0:QT, c0:c1], k_ref[:, c0:c1], (((1,), (1,)), ((), ())),
                preferred_element_type=jnp.float32) * SCALE + bias0[...]
            w = jnp.exp(s)
            r = 1.0 / jnp.sum(w, axis=1, keepdims=True)
            p = (w * r).astype(jnp.bfloat16)
            ctx_ref[0:QT, c0:c1] = lax.dot_general(
                p, v_ref[:, c0:c1], (((1,), (0,)), ((), ())),
                preferred_element_type=jnp.float32).astype(jnp.bfloat16)

        kig = lax.broadcasted_iota(jnp.int32, (QT, 128), 1)
        biasg[...] = jnp.where(kig < 32, 0.0, -1e9)
        WB = QT + 2 * 128
        for t in range(1, NQT):
            w0 = min(t * QT - 128, SKV - WB)
            qib = lax.broadcasted_iota(jnp.int32, (QT, WB), 0) + t * QT
            kib = lax.broadcasted_iota(jnp.int32, (QT, WB), 1) + w0
            biasb[...] = jnp.where(jnp.abs(qib - kib) <= 128, 0.0, -1e9)

            for j in range(HP):
                c0, c1 = j * DH, (j + 1) * DH
                qh = q_ref[t * QT:(t + 1) * QT, c0:c1]
                sb = lax.dot_general(
                    qh, k_ref[w0:w0 + WB, c0:c1], (((1,), (1,)), ((), ())),
                    preferred_element_type=jnp.float32) * SCALE + biasb[...]
                sg = lax.dot_general(
                    qh, k_ref[0:128, c0:c1], (((1,), (1,)), ((), ())),
                    preferred_element_type=jnp.float32) * SCALE + biasg[...]
                wb = jnp.exp(sb)
                wg = jnp.exp(sg)
                r = 1.0 / (jnp.sum(wb, axis=1, keepdims=True)
                           + jnp.sum(wg, axis=1, keepdims=True))
                ctx = lax.dot_general(
                    (wb * r).astype(jnp.bfloat16), v_ref[w0:w0 + WB, c0:c1],
                    (((1,), (0,)), ((), ())),
                    preferred_element_type=jnp.float32)
                ctx = ctx + lax.dot_general(
                    (wg * r).astype(jnp.bfloat16), v_ref[0:128, c0:c1],
                    (((1,), (0,)), ((), ())),
                    preferred_element_type=jnp.float32)
                ctx_ref[t * QT:(t + 1) * QT, c0:c1] = ctx.astype(jnp.bfloat16)

        for rt in range(NQT):
            r0 = rt * QT
            acc_bf[r0:r0 + QT, :] = lax.dot_general(
                ctx_ref[r0:r0 + QT, :], wo_ref[...],
                (((1,), (0,)), ((), ())),
                preferred_element_type=jnp.float32).astype(jnp.bfloat16)

        L, R = (0, HC), (HC, DM)

        def exchange(partner, sends, col, sems, rsb=None, base=0):
            c0, c1 = col
            rs = []
            for i, (sid, rid) in enumerate(sends):
                slot = base + i
                dst = (rsb.at[slot] if rsb is not None
                       else acc_bf.at[pl.ds(sid * CH, CH), c0:c1])
                r = pltpu.make_async_remote_copy(
                    src_ref=acc_bf.at[pl.ds(sid * CH, CH), c0:c1],
                    dst_ref=dst,
                    send_sem=sems[0].at[slot], recv_sem=sems[1].at[slot],
                    device_id=(partner,),
                    device_id_type=pl.DeviceIdType.MESH)
                r.start()
                rs.append(r)
            return rs

        def accumulate(sends, col, rsb, base):
            c0, c1 = col
            for i, (_, rid) in enumerate(sends):
                off = rid * CH
                acc_bf[pl.ds(off, CH), c0:c1] = (
                    acc_bf[pl.ds(off, CH), c0:c1] + rsb[base + i])

        b2_ = [(0, 0), (1, 0), (0, 1), (1, 1)]
        rs_L = [
            (xp, [((1 - cx) + 2 * b1 + 4 * b2, cx + 2 * b1 + 4 * b2)
                  for b1, b2 in b2_]),
            (yp, [(cx + 2 * (1 - cy) + 4 * b2, cx + 2 * cy + 4 * b2)
                  for b2 in (0, 1)]),
            (zp, [(cx + 2 * cy + 4 * (1 - cz), cid)]),
        ]
        rs_R = [
            (yp, [(bx + 2 * (1 - cy) + 4 * bz, bx + 2 * cy + 4 * bz)
                  for bx, bz in b2_]),
            (zp, [(bx + 2 * cy + 4 * (1 - cz), bx + 2 * cy + 4 * cz)
                  for bx in (0, 1)]),
            (xp, [((1 - cx) + 2 * cy + 4 * cz, cid)]),
        ]
        ag_L = [
            (zp, [(cid, 0)]),
            (yp, [(cx + 2 * cy + 4 * b2, 0) for b2 in (0, 1)]),
            (xp, [(cx + 2 * b1 + 4 * b2, 0) for b1, b2 in b2_]),
        ]
        ag_R = [
            (xp, [(cid, 0)]),
            (zp, [(bx + 2 * cy + 4 * cz, 0) for bx in (0, 1)]),
            (yp, [(bx + 2 * cy + 4 * bz, 0) for bx, bz in b2_]),
        ]
        bases = [0, 4, 6]
        ag_bases = [7, 8, 10]

        for st in range(3):
            pl_, sl = rs_L[st]
            pm_, sm_ = rs_R[st]
            rl = exchange(pl_, sl, L, (send_p, recv_p), rsb_p, bases[st])
            rm = exchange(pm_, sm_, R, (send_m, recv_m), rsb_m, bases[st])
            for r in rl + rm:
                r.wait()
            accumulate(sl, L, rsb_p, bases[st])
            accumulate(sm_, R, rsb_m, bases[st])

        for st in range(3):
            pl_, sl = ag_L[st]
            pm_, sm_ = ag_R[st]
            rl = exchange(pl_, sl, L, (send_p, recv_p), None, ag_bases[st])
            rm = exchange(pm_, sm_, R, (send_m, recv_m), None, ag_bases[st])
            for r in rl + rm:
                r.wait()

        out_ref[0] = acc_bf[...].astype(jnp.float32)

    kwargs = {}
    if _INTERPRET:
        kwargs["interpret"] = pltpu.InterpretParams()

    return pl.pallas_call(
        body,
        out_shape=jax.ShapeDtypeStruct((1, SQ, DM), jnp.float32),
        in_specs=[
            pl.BlockSpec(memory_space=pltpu.MemorySpace.VMEM),
            pl.BlockSpec(memory_space=pltpu.MemorySpace.VMEM),
            pl.BlockSpec(memory_space=pltpu.MemorySpace.VMEM),
            pl.BlockSpec(memory_space=pltpu.MemorySpace.VMEM),
            pl.BlockSpec(memory_space=pltpu.MemorySpace.VMEM),
        ],
        out_specs=pl.BlockSpec(memory_space=pltpu.MemorySpace.VMEM),
        scratch_shapes=[
            pltpu.VMEM((SQ, DM), jnp.bfloat16),
            pltpu.VMEM((SQ, DM), jnp.bfloat16),
            pltpu.VMEM((SQ, DM), jnp.bfloat16),
            pltpu.VMEM((N_DEV - 1, CH, HC), jnp.bfloat16),
            pltpu.VMEM((N_DEV - 1, CH, HC), jnp.bfloat16),
            pltpu.VMEM((QT, SKV), jnp.float32),
            pltpu.VMEM((QT, QT + 256), jnp.float32),
            pltpu.VMEM((QT, 128), jnp.float32),
            pltpu.SemaphoreType.DMA((N_HOPS,)),
            pltpu.SemaphoreType.DMA((N_HOPS,)),
            pltpu.SemaphoreType.DMA((N_HOPS,)),
            pltpu.SemaphoreType.DMA((N_HOPS,)),
        ],
        compiler_params=pltpu.CompilerParams(
            collective_id=0,
            vmem_limit_bytes=62 * 1024 * 1024,
        ),
        **kwargs,
    )(xb, wqb, kb, vb, wob)


# device time: 136378 ns/iter; 1.1211x vs baseline; 1.0174x over previous
import os

import jax
import jax.numpy as jnp
from jax import lax
from jax.experimental import pallas as pl
from jax.experimental.pallas import tpu as pltpu

N_DEV = 8
HP = 8
DH = 128
SQ = 2048
SKV = 2048
DM = 1024
HC = DM // 2
CH = SQ // N_DEV
QT = 512
NQT = SQ // QT
SCALE = 0.08838834764831843
N_HOPS = 2 * (N_DEV - 1)

_INTERPRET = os.environ.get("SCBAND_INTERPRET") == "1"


def kernel(x, Wq, K_ext, V_ext, Wo):
    my_out = lax.axis_index("i")
    xb = x.astype(jnp.bfloat16)
    wqb = Wq.astype(jnp.bfloat16)
    wob = Wo.astype(jnp.bfloat16)
    kb = lax.dynamic_slice(
        K_ext[0], (0, my_out * HP, 0), (SKV, HP, DH)
    ).astype(jnp.bfloat16).reshape(SKV, HP * DH)
    vb = lax.dynamic_slice(
        V_ext[0], (0, my_out * HP, 0), (SKV, HP, DH)
    ).astype(jnp.bfloat16).reshape(SKV, HP * DH)

    def body(x_ref, wq_ref, k_ref, v_ref, wo_ref, out_ref,
             q_ref, ctx_ref, acc_bf, rsb_p, rsb_m,
             bias0, biasB, biasb, biasg,
             send_p, recv_p, send_m, recv_m):
        my = lax.axis_index("i")
        p_ = lax.rem(my, 4)
        cz = my // 4
        cy = p_ // 2
        cx = (p_ % 2) ^ cy

        def ring_idx(x, y, z):
            return 4 * z + 2 * y + (x ^ y)

        xp = ring_idx(1 - cx, cy, cz)
        yp = ring_idx(cx, 1 - cy, cz)
        zp = ring_idx(cx, cy, 1 - cz)
        cid = cx + 2 * cy + 4 * cz

        barrier = pltpu.get_barrier_semaphore()
        for nbr in (xp, yp, zp):
            pl.semaphore_signal(barrier, inc=1, device_id=(nbr,),
                                device_id_type=pl.DeviceIdType.MESH)
        pl.semaphore_wait(barrier, 3)

        for rt in range(NQT):
            r0 = rt * QT
            q_ref[r0:r0 + QT, :] = lax.dot_general(
                x_ref[0, r0:r0 + QT, :], wq_ref[...],
                (((1,), (0,)), ((), ())),
                preferred_element_type=jnp.float32).astype(jnp.bfloat16)

        WB = QT + 2 * 128
        RA = 128
        RB = QT - RA
        qi = lax.broadcasted_iota(jnp.int32, (RA, SKV), 0)
        ki = lax.broadcasted_iota(jnp.int32, (RA, SKV), 1)
        bias0[...] = jnp.where(
            (jnp.abs(qi - ki) <= 128) | (ki < 32) | (qi < 32), 0.0, -1e9)
        qiB = lax.broadcasted_iota(jnp.int32, (RB, WB), 0) + RA
        kiB = lax.broadcasted_iota(jnp.int32, (RB, WB), 1)
        biasB[...] = jnp.where(
            (jnp.abs(qiB - kiB) <= 128) | (kiB < 32), 0.0, -1e9)

        for j in range(HP):
            c0, c1 = j * DH, (j + 1) * DH
            s = lax.dot_general(
                q_ref[0:RA, c0:c1], k_ref[:, c0:c1], (((1,), (1,)), ((), ())),
                preferred_element_type=jnp.float32) * SCALE + bias0[...]
            w = jnp.exp(s)
            r = 1.0 / jnp.sum(w, axis=1, keepdims=True)
            p = (w * r).astype(jnp.bfloat16)
            ctx_ref[0:RA, c0:c1] = lax.dot_general(
                p, v_ref[:, c0:c1], (((1,), (0,)), ((), ())),
                preferred_element_type=jnp.float32).astype(jnp.bfloat16)

            sB = lax.dot_general(
                q_ref[RA:QT, c0:c1], k_ref[0:WB, c0:c1],
                (((1,), (1,)), ((), ())),
                preferred_element_type=jnp.float32) * SCALE + biasB[...]
            wB = jnp.exp(sB)
            rB = 1.0 / jnp.sum(wB, axis=1, keepdims=True)
            pB = (wB * rB).astype(jnp.bfloat16)
            ctx_ref[RA:QT, c0:c1] = lax.dot_general(
                pB, v_ref[0:WB, c0:c1], (((1,), (0,)), ((), ())),
                preferred_element_type=jnp.float32).astype(jnp.bfloat16)

        kig = lax.broadcasted_iota(jnp.int32, (QT, 128), 1)
        biasg[...] = jnp.where(kig < 32, 0.0, -1e9)
        for t in range(1, NQT):
            w0 = min(t * QT - 128, SKV - WB)
            qib = lax.broadcasted_iota(jnp.int32, (QT, WB), 0) + t * QT
            kib = lax.broadcasted_iota(jnp.int32, (QT, WB), 1) + w0
            biasb[...] = jnp.where(jnp.abs(qib - kib) <= 128, 0.0, -1e9)

            for j in range(HP):
                c0, c1 = j * DH, (j + 1) * DH
                qh = q_ref[t * QT:(t + 1) * QT, c0:c1]
                sb = lax.dot_general(
                    qh, k_ref[w0:w0 + WB, c0:c1], (((1,), (1,)), ((), ())),
                    preferred_element_type=jnp.float32) * SCALE + biasb[...]
                sg = lax.dot_general(
                    qh, k_ref[0:128, c0:c1], (((1,), (1,)), ((), ())),
                    preferred_element_type=jnp.float32) * SCALE + biasg[...]
                wb = jnp.exp(sb)
                wg = jnp.exp(sg)
                r = 1.0 / (jnp.sum(wb, axis=1, keepdims=True)
                           + jnp.sum(wg, axis=1, keepdims=True))
                ctx = lax.dot_general(
                    (wb * r).astype(jnp.bfloat16), v_ref[w0:w0 + WB, c0:c1],
                    (((1,), (0,)), ((), ())),
                    preferred_element_type=jnp.float32)
                ctx = ctx + lax.dot_general(
                    (wg * r).astype(jnp.bfloat16), v_ref[0:128, c0:c1],
                    (((1,), (0,)), ((), ())),
                    preferred_element_type=jnp.float32)
                ctx_ref[t * QT:(t + 1) * QT, c0:c1] = ctx.astype(jnp.bfloat16)

        for rt in range(NQT):
            r0 = rt * QT
            acc_bf[r0:r0 + QT, :] = lax.dot_general(
                ctx_ref[r0:r0 + QT, :], wo_ref[...],
                (((1,), (0,)), ((), ())),
                preferred_element_type=jnp.float32).astype(jnp.bfloat16)

        L, R = (0, HC), (HC, DM)

        def exchange(partner, sends, col, sems, rsb=None, base=0):
            c0, c1 = col
            rs = []
            for i, (sid, rid) in enumerate(sends):
                slot = base + i
                dst = (rsb.at[slot] if rsb is not None
                       else acc_bf.at[pl.ds(sid * CH, CH), c0:c1])
                r = pltpu.make_async_remote_copy(
                    src_ref=acc_bf.at[pl.ds(sid * CH, CH), c0:c1],
                    dst_ref=dst,
                    send_sem=sems[0].at[slot], recv_sem=sems[1].at[slot],
                    device_id=(partner,),
                    device_id_type=pl.DeviceIdType.MESH)
                r.start()
                rs.append(r)
            return rs

        def accumulate(sends, col, rsb, base):
            c0, c1 = col
            for i, (_, rid) in enumerate(sends):
                off = rid * CH
                acc_bf[pl.ds(off, CH), c0:c1] = (
                    acc_bf[pl.ds(off, CH), c0:c1] + rsb[base + i])

        b2_ = [(0, 0), (1, 0), (0, 1), (1, 1)]
        rs_L = [
            (xp, [((1 - cx) + 2 * b1 + 4 * b2, cx + 2 * b1 + 4 * b2)
                  for b1, b2 in b2_]),
            (yp, [(cx + 2 * (1 - cy) + 4 * b2, cx + 2 * cy + 4 * b2)
                  for b2 in (0, 1)]),
            (zp, [(cx + 2 * cy + 4 * (1 - cz), cid)]),
        ]
        rs_R = [
            (yp, [(bx + 2 * (1 - cy) + 4 * bz, bx + 2 * cy + 4 * bz)
                  for bx, bz in b2_]),
            (zp, [(bx + 2 * cy + 4 * (1 - cz), bx + 2 * cy + 4 * cz)
                  for bx in (0, 1)]),
            (xp, [((1 - cx) + 2 * cy + 4 * cz, cid)]),
        ]
        ag_L = [
            (zp, [(cid, 0)]),
            (yp, [(cx + 2 * cy + 4 * b2, 0) for b2 in (0, 1)]),
            (xp, [(cx + 2 * b1 + 4 * b2, 0) for b1, b2 in b2_]),
        ]
        ag_R = [
            (xp, [(cid, 0)]),
            (zp, [(bx + 2 * cy + 4 * cz, 0) for bx in (0, 1)]),
            (yp, [(bx + 2 * cy + 4 * bz, 0) for bx, bz in b2_]),
        ]
        bases = [0, 4, 6]
        ag_bases = [7, 8, 10]

        for st in range(3):
            pl_, sl = rs_L[st]
            pm_, sm_ = rs_R[st]
            rl = exchange(pl_, sl, L, (send_p, recv_p), rsb_p, bases[st])
            rm = exchange(pm_, sm_, R, (send_m, recv_m), rsb_m, bases[st])
            for r in rl + rm:
                r.wait()
            accumulate(sl, L, rsb_p, bases[st])
            accumulate(sm_, R, rsb_m, bases[st])

        for st in range(3):
            pl_, sl = ag_L[st]
            pm_, sm_ = ag_R[st]
            rl = exchange(pl_, sl, L, (send_p, recv_p), None, ag_bases[st])
            rm = exchange(pm_, sm_, R, (send_m, recv_m), None, ag_bases[st])
            for r in rl + rm:
                r.wait()

        out_ref[0] = acc_bf[...].astype(jnp.float32)

    kwargs = {}
    if _INTERPRET:
        kwargs["interpret"] = pltpu.InterpretParams()

    return pl.pallas_call(
        body,
        out_shape=jax.ShapeDtypeStruct((1, SQ, DM), jnp.float32),
        in_specs=[
            pl.BlockSpec(memory_space=pltpu.MemorySpace.VMEM),
            pl.BlockSpec(memory_space=pltpu.MemorySpace.VMEM),
            pl.BlockSpec(memory_space=pltpu.MemorySpace.VMEM),
            pl.BlockSpec(memory_space=pltpu.MemorySpace.VMEM),
            pl.BlockSpec(memory_space=pltpu.MemorySpace.VMEM),
        ],
        out_specs=pl.BlockSpec(memory_space=pltpu.MemorySpace.VMEM),
        scratch_shapes=[
            pltpu.VMEM((SQ, DM), jnp.bfloat16),
            pltpu.VMEM((SQ, DM), jnp.bfloat16),
            pltpu.VMEM((SQ, DM), jnp.bfloat16),
            pltpu.VMEM((N_DEV - 1, CH, HC), jnp.bfloat16),
            pltpu.VMEM((N_DEV - 1, CH, HC), jnp.bfloat16),
            pltpu.VMEM((128, SKV), jnp.float32),
            pltpu.VMEM((QT - 128, QT + 256), jnp.float32),
            pltpu.VMEM((QT, QT + 256), jnp.float32),
            pltpu.VMEM((QT, 128), jnp.float32),
            pltpu.SemaphoreType.DMA((N_HOPS,)),
            pltpu.SemaphoreType.DMA((N_HOPS,)),
            pltpu.SemaphoreType.DMA((N_HOPS,)),
            pltpu.SemaphoreType.DMA((N_HOPS,)),
        ],
        compiler_params=pltpu.CompilerParams(
            collective_id=0,
            vmem_limit_bytes=62 * 1024 * 1024,
        ),
        **kwargs,
    )(xb, wqb, kb, vb, wob)
